# Initial kernel scaffold; baseline (speedup 1.0000x reference)
import jax
import jax.numpy as jnp
from jax.experimental import pallas as pl


def kernel(features, points):
    raise NotImplementedError("write your pallas kernel here")


# speedup vs baseline: 8.5887x; 8.5887x over previous
import functools

import jax
import jax.numpy as jnp
from jax import lax
from jax.experimental import pallas as pl

N = 2048
K = 16
ROWS = 256


def _knn_body(points_rows_ref, points_t_ref, out_ref):
    d2 = None
    for c in range(3):
        rows_c = points_rows_ref[0, :, c][:, None]
        cols_c = points_t_ref[0, c, :][None, :]
        diff = rows_c - cols_c
        sq = diff * diff
        d2 = sq if d2 is None else d2 + sq
    dist = jnp.sqrt(d2)

    col_iota = lax.broadcasted_iota(jnp.int32, (ROWS, N), 1)
    for it in range(K + 1):
        m = jnp.min(dist, axis=1, keepdims=True)
        is_min = dist == m
        idx = jnp.min(jnp.where(is_min, col_iota, N), axis=1, keepdims=True)
        if it > 0:
            out_ref[0, :, it - 1] = idx[:, 0]
        dist = jnp.where(col_iota == idx, jnp.inf, dist)


@jax.jit
def kernel(features, points):
    del features
    b, n, _ = points.shape
    points_t = jnp.transpose(points, (0, 2, 1))
    grid = (b, n // ROWS)
    topk = pl.pallas_call(
        _knn_body,
        grid=grid,
        in_specs=[
            pl.BlockSpec((1, ROWS, 3), lambda bi, ri: (bi, ri, 0)),
            pl.BlockSpec((1, 3, N), lambda bi, ri: (bi, 0, 0)),
        ],
        out_specs=pl.BlockSpec((1, ROWS, K), lambda bi, ri: (bi, ri, 0)),
        out_shape=jax.ShapeDtypeStruct((b, n, K), jnp.int32),
    )(points, points_t)
    batch_ids = jnp.broadcast_to(
        jnp.arange(b, dtype=jnp.int32).reshape(b, 1, 1, 1), (b, n, K, 1)
    )
    return jnp.concatenate([batch_ids, topk[..., None]], axis=3)
